# baseline (speedup 1.0000x reference)
import jax
import jax.numpy as jnp
import numpy as np
from math import sqrt
from jax.experimental import pallas as pl
from jax.experimental.pallas import tpu as pltpu

_MAPS = (38, 19, 10, 5, 3, 1)
_NA = (4, 6, 6, 6, 4, 4)
_RATIOS = (1, 2, 3)
_SCALES = (21, 45, 99, 153, 207, 261, 315)

N_ANCHORS = 8732
N_CLASSES = 21
B_IMG = 32
G_PER = 8
NI = 8
N_GRP = B_IMG // NI


def _build_default_boxes():
    m = 6
    fig = 300.0
    sb = np.array(_SCALES, dtype=np.float64) / fig
    sbh = [sqrt(sb[k] * sb[k + 1]) for k in range(m)]
    sb6 = sb[:m]
    whs = []
    for r in _RATIOS:
        rs = sqrt(r)
        if rs == 1.0:
            whs += [np.array([sb6, sb6]), np.array([sbh, sbh])]
        else:
            w = sb6 * rs
            h = sb6 / rs
            whs += [np.array([w, h]), np.array([h, w])]
    whs = np.array(whs).transpose([2, 1, 0]).clip(0, 1)
    mm = max(_MAPS)
    ax = np.arange(mm, dtype=np.float64) + 0.5
    cx = np.broadcast_to(ax.reshape(1, mm), (mm, mm))
    cy = np.broadcast_to(ax.reshape(mm, 1), (mm, mm))
    org = np.stack([cx, cy, np.ones((mm, mm)), np.ones((mm, mm))], 0)
    parts = []
    for na, ms, bw in zip(_NA, _MAPS, whs):
        fm = org[:, :ms, :ms].reshape(4, -1).copy()
        fm[:2] /= ms
        for idx in range(na):
            f = fm.copy()
            f[-2:] = bw[:, idx:idx + 1]
            parts.append(f)
    xywh = np.clip(np.concatenate(parts, axis=-1), 0, 1)
    x, y, w, h = xywh
    ltrb = np.stack([x - w / 2.0, y - h / 2.0, x + w / 2.0, y + h / 2.0])
    return np.concatenate([xywh, ltrb], axis=0).astype(np.float32)

_DB_ALL = _build_default_boxes()


def _ssd_body(yp_ref, yt_ref, db_ref, out_ref, buf, sem):
    i = pl.program_id(0)

    def copies(grp, slot):
        return [pltpu.make_async_copy(
            yp_ref.at[pl.ds(grp * NI, NI), ch, 0],
            buf.at[slot, ch],
            sem.at[slot]) for ch in range(25)]

    @pl.when(i == 0)
    def _():
        for c in copies(0, jnp.int32(0)):
            c.start()

    @pl.when(i + 1 < N_GRP)
    def _():
        for c in copies(i + 1, jax.lax.rem(i + 1, 2)):
            c.start()

    slot = jax.lax.rem(i, 2)
    for c in copies(i, slot):
        c.wait()

    pred = buf[slot]
    yt = yt_ref[...]
    db = db_ref[...]

    dbx, dby, dbw, dbh = db[0:1], db[1:2], db[2:3], db[3:4]
    dl, dt, dr, dbt = db[4:5], db[5:6], db[6:7], db[7:8]
    a1 = (dr - dl) * (dbt - dt)
    lane = jax.lax.broadcasted_iota(jnp.int32, (NI, N_ANCHORS), 1)

    best = None
    gcls = gcx = gcy = gw = gh = None
    for g in range(G_PER):
        ytg = yt[g]
        cx, cy = ytg[:, 2:3], ytg[:, 3:4]
        w, h = ytg[:, 4:5], ytg[:, 5:6]
        gl, gr = cx - w * 0.5, cx + w * 0.5
        gt_, gb = cy - h * 0.5, cy + h * 0.5
        il = jnp.maximum(dl, gl)
        it = jnp.minimum(dr, gr)
        inter = (jnp.maximum(it - il, 0.0) *
                 jnp.maximum(jnp.minimum(dbt, gb) - jnp.maximum(dt, gt_), 0.0))
        a2 = (gr - gl) * (gb - gt_)
        iou = inter / (a1 + a2 - inter)
        rowmax = jnp.max(iou, axis=1, keepdims=True)
        dbox = jnp.min(jnp.where(iou == rowmax, lane, jnp.int32(2**30)),
                       axis=1, keepdims=True)
        iou = jnp.where(lane == dbox, 2.0, iou)
        if g == 0:
            best = iou
            full = (NI, N_ANCHORS)
            gcls = jnp.broadcast_to(ytg[:, 1:2], full)
            gcx = jnp.broadcast_to(cx, full)
            gcy = jnp.broadcast_to(cy, full)
            gw = jnp.broadcast_to(w, full)
            gh = jnp.broadcast_to(h, full)
        else:
            take = iou > best
            best = jnp.where(take, iou, best)
            gcls = jnp.where(take, ytg[:, 1:2], gcls)
            gcx = jnp.where(take, cx, gcx)
            gcy = jnp.where(take, cy, gcy)
            gw = jnp.where(take, w, gw)
            gh = jnp.where(take, h, gh)
    pos_mask = best > 0.5
    posf = pos_mask.astype(jnp.float32)
    n_pos = jnp.sum(posf, axis=1, keepdims=True)

    hats = ((gcx - dbx) / dbw, (gcy - dby) / dbh,
            jnp.log(gw / dbw), jnp.log(gh / dbh))
    loc_sum = jnp.zeros((NI, 1), jnp.float32)
    for r in range(4):
        d = jnp.abs(pred[r] - hats[r])
        sl = jnp.where(d < 1.0, 0.5 * d * d, d - 0.5)
        loc_sum = loc_sum + jnp.sum(sl * posf, axis=1, keepdims=True)
    loc_loss = loc_sum / (4.0 * n_pos)

    x = pred[4:25]
    xmax = jnp.max(x, axis=0)
    sumexp = jnp.sum(jnp.exp(x - xmax[None]), axis=0)
    lse = xmax + jnp.log(sumexp)

    gci = gcls.astype(jnp.int32)
    m0 = (gci & 1) > 0
    m1 = (gci & 2) > 0
    m2 = (gci & 4) > 0
    m3 = (gci & 8) > 0
    m4 = (gci & 16) > 0
    t = [jnp.where(m0, pred[4 + 2 * j + 1], pred[4 + 2 * j])
         for j in range(10)] + [pred[24]]
    u = [jnp.where(m1, t[2 * j + 1], t[2 * j]) for j in range(5)] + [t[10]]
    v = [jnp.where(m2, u[2 * j + 1], u[2 * j]) for j in range(3)]
    w0 = jnp.where(m3, v[1], v[0])
    xg = jnp.where(m4, v[2], w0)
    conf_pos_mean = (jnp.sum(jnp.where(pos_mask, lse - xg, 0.0),
                             axis=1, keepdims=True) / n_pos)

    neg = lse - pred[4]
    cand = jnp.where(pos_mask, -1.0, neg)
    vb = jax.lax.bitcast_convert_type(cand, jnp.int32)
    m_i = 3 * jnp.sum(pos_mask.astype(jnp.int32), axis=1, keepdims=True)
    hi0 = jnp.max(vb, axis=1, keepdims=True) + 1

    def search_body(_, carry):
        lo, hi = carry
        mid = lo + (hi - lo) // 2
        cnt = jnp.sum((vb >= mid).astype(jnp.int32), axis=1, keepdims=True)
        ge = cnt >= m_i
        return jnp.where(ge, mid, lo), jnp.where(ge, hi, mid)

    lo_f, _ = jax.lax.fori_loop(
        0, 32, search_body, (jnp.zeros((NI, 1), jnp.int32), hi0))
    vm = jax.lax.bitcast_convert_type(lo_f, jnp.float32)
    gt_mask = vb > lo_f
    c_gt = jnp.sum(gt_mask.astype(jnp.float32), axis=1, keepdims=True)
    s_gt = jnp.sum(jnp.where(gt_mask, cand, 0.0), axis=1, keepdims=True)
    mf = m_i.astype(jnp.float32)
    top_mean = (s_gt + (mf - c_gt) * vm) / mf

    loss = loc_loss + conf_pos_mean + top_mean
    img = (i * NI + jax.lax.broadcasted_iota(jnp.int32, (NI, 1), 0))
    w = img.astype(jnp.float32) * (1.0 / B_IMG)
    out_ref[...] = jnp.broadcast_to(w * loss, (NI, 128))[None]


def kernel(y_pred, y_true):
    ytt = jnp.transpose(y_true.reshape(B_IMG, G_PER, 6), (1, 0, 2))
    yp4 = y_pred.reshape(B_IMG, 25, 1, N_ANCHORS)
    db = jnp.asarray(_DB_ALL)
    out = pl.pallas_call(
        _ssd_body,
        grid=(N_GRP,),
        in_specs=[
            pl.BlockSpec(memory_space=pl.ANY),
            pl.BlockSpec((G_PER, NI, 6), lambda i: (0, i, 0)),
            pl.BlockSpec((G_PER, N_ANCHORS), lambda i: (0, 0)),
        ],
        out_specs=pl.BlockSpec((1, NI, 128), lambda i: (i, 0, 0)),
        out_shape=jax.ShapeDtypeStruct((N_GRP, NI, 128), jnp.float32),
        scratch_shapes=[
            pltpu.VMEM((2, 25, NI, N_ANCHORS), jnp.float32),
            pltpu.SemaphoreType.DMA((2,)),
        ],
        compiler_params=pltpu.CompilerParams(
            dimension_semantics=("arbitrary",)),
    )(yp4, ytt, db)
    return jnp.sum(out[:, :, 0])


# speedup vs baseline: 1.6999x; 1.6999x over previous
import jax
import jax.numpy as jnp
import numpy as np
from math import sqrt
from jax.experimental import pallas as pl
from jax.experimental.pallas import tpu as pltpu

_MAPS = (38, 19, 10, 5, 3, 1)
_NA = (4, 6, 6, 6, 4, 4)
_RATIOS = (1, 2, 3)
_SCALES = (21, 45, 99, 153, 207, 261, 315)

N_ANCHORS = 8732
N_CLASSES = 21
B_IMG = 32
G_PER = 8
NI = 8
N_GRP = B_IMG // NI


def _build_default_boxes():
    m = 6
    fig = 300.0
    sb = np.array(_SCALES, dtype=np.float64) / fig
    sbh = [sqrt(sb[k] * sb[k + 1]) for k in range(m)]
    sb6 = sb[:m]
    whs = []
    for r in _RATIOS:
        rs = sqrt(r)
        if rs == 1.0:
            whs += [np.array([sb6, sb6]), np.array([sbh, sbh])]
        else:
            w = sb6 * rs
            h = sb6 / rs
            whs += [np.array([w, h]), np.array([h, w])]
    whs = np.array(whs).transpose([2, 1, 0]).clip(0, 1)
    mm = max(_MAPS)
    ax = np.arange(mm, dtype=np.float64) + 0.5
    cx = np.broadcast_to(ax.reshape(1, mm), (mm, mm))
    cy = np.broadcast_to(ax.reshape(mm, 1), (mm, mm))
    org = np.stack([cx, cy, np.ones((mm, mm)), np.ones((mm, mm))], 0)
    parts = []
    for na, ms, bw in zip(_NA, _MAPS, whs):
        fm = org[:, :ms, :ms].reshape(4, -1).copy()
        fm[:2] /= ms
        for idx in range(na):
            f = fm.copy()
            f[-2:] = bw[:, idx:idx + 1]
            parts.append(f)
    xywh = np.clip(np.concatenate(parts, axis=-1), 0, 1)
    x, y, w, h = xywh
    ltrb = np.stack([x - w / 2.0, y - h / 2.0, x + w / 2.0, y + h / 2.0])
    return np.concatenate([xywh, ltrb], axis=0).astype(np.float32)

_DB_ALL = _build_default_boxes()


def _ssd_body(pred_ref, yt_ref, db_ref, out_ref):
    i = pl.program_id(0)
    pred = pred_ref[...]
    yt = yt_ref[...]
    db = db_ref[...]

    dbx, dby, dbw, dbh = db[0:1], db[1:2], db[2:3], db[3:4]
    dl, dt, dr, dbt = db[4:5], db[5:6], db[6:7], db[7:8]
    a1 = (dr - dl) * (dbt - dt)
    lane = jax.lax.broadcasted_iota(jnp.int32, (NI, N_ANCHORS), 1)

    best = None
    gcls = gcx = gcy = gw = gh = None
    for g in range(G_PER):
        ytg = yt[g]
        cx, cy = ytg[:, 2:3], ytg[:, 3:4]
        w, h = ytg[:, 4:5], ytg[:, 5:6]
        gl, gr = cx - w * 0.5, cx + w * 0.5
        gt_, gb = cy - h * 0.5, cy + h * 0.5
        il = jnp.maximum(dl, gl)
        it = jnp.minimum(dr, gr)
        inter = (jnp.maximum(it - il, 0.0) *
                 jnp.maximum(jnp.minimum(dbt, gb) - jnp.maximum(dt, gt_), 0.0))
        a2 = (gr - gl) * (gb - gt_)
        iou = inter / (a1 + a2 - inter)
        rowmax = jnp.max(iou, axis=1, keepdims=True)
        dbox = jnp.min(jnp.where(iou == rowmax, lane, jnp.int32(2**30)),
                       axis=1, keepdims=True)
        iou = jnp.where(lane == dbox, 2.0, iou)
        if g == 0:
            best = iou
            full = (NI, N_ANCHORS)
            gcls = jnp.broadcast_to(ytg[:, 1:2], full)
            gcx = jnp.broadcast_to(cx, full)
            gcy = jnp.broadcast_to(cy, full)
            gw = jnp.broadcast_to(w, full)
            gh = jnp.broadcast_to(h, full)
        else:
            take = iou > best
            best = jnp.where(take, iou, best)
            gcls = jnp.where(take, ytg[:, 1:2], gcls)
            gcx = jnp.where(take, cx, gcx)
            gcy = jnp.where(take, cy, gcy)
            gw = jnp.where(take, w, gw)
            gh = jnp.where(take, h, gh)
    pos_mask = best > 0.5
    posf = pos_mask.astype(jnp.float32)
    n_pos = jnp.sum(posf, axis=1, keepdims=True)

    hats = ((gcx - dbx) / dbw, (gcy - dby) / dbh,
            jnp.log(gw / dbw), jnp.log(gh / dbh))
    loc_sum = jnp.zeros((NI, 1), jnp.float32)
    for r in range(4):
        d = jnp.abs(pred[r] - hats[r])
        sl = jnp.where(d < 1.0, 0.5 * d * d, d - 0.5)
        loc_sum = loc_sum + jnp.sum(sl * posf, axis=1, keepdims=True)
    loc_loss = loc_sum / (4.0 * n_pos)

    x = pred[4:25]
    xmax = jnp.max(x, axis=0)
    sumexp = jnp.sum(jnp.exp(x - xmax[None]), axis=0)
    lse = xmax + jnp.log(sumexp)

    gci = gcls.astype(jnp.int32)
    m0 = (gci & 1) > 0
    m1 = (gci & 2) > 0
    m2 = (gci & 4) > 0
    m3 = (gci & 8) > 0
    m4 = (gci & 16) > 0
    t = [jnp.where(m0, pred[4 + 2 * j + 1], pred[4 + 2 * j])
         for j in range(10)] + [pred[24]]
    u = [jnp.where(m1, t[2 * j + 1], t[2 * j]) for j in range(5)] + [t[10]]
    v = [jnp.where(m2, u[2 * j + 1], u[2 * j]) for j in range(3)]
    w0 = jnp.where(m3, v[1], v[0])
    xg = jnp.where(m4, v[2], w0)
    conf_pos_mean = (jnp.sum(jnp.where(pos_mask, lse - xg, 0.0),
                             axis=1, keepdims=True) / n_pos)

    neg = lse - pred[4]
    cand = jnp.where(pos_mask, -1.0, neg)
    vb = jax.lax.bitcast_convert_type(cand, jnp.int32)
    m_i = 3 * jnp.sum(pos_mask.astype(jnp.int32), axis=1, keepdims=True)
    hi0 = jnp.max(vb, axis=1, keepdims=True) + 1

    def search_body(_, carry):
        lo, hi = carry
        mid = lo + (hi - lo) // 2
        cnt = jnp.sum((vb >= mid).astype(jnp.int32), axis=1, keepdims=True)
        ge = cnt >= m_i
        return jnp.where(ge, mid, lo), jnp.where(ge, hi, mid)

    lo_f, _ = jax.lax.fori_loop(
        0, 32, search_body, (jnp.zeros((NI, 1), jnp.int32), hi0))
    vm = jax.lax.bitcast_convert_type(lo_f, jnp.float32)
    gt_mask = vb > lo_f
    c_gt = jnp.sum(gt_mask.astype(jnp.float32), axis=1, keepdims=True)
    s_gt = jnp.sum(jnp.where(gt_mask, cand, 0.0), axis=1, keepdims=True)
    mf = m_i.astype(jnp.float32)
    top_mean = (s_gt + (mf - c_gt) * vm) / mf

    loss = loc_loss + conf_pos_mean + top_mean
    img = (i * NI + jax.lax.broadcasted_iota(jnp.int32, (NI, 1), 0))
    w = img.astype(jnp.float32) * (1.0 / B_IMG)
    out_ref[...] = jnp.broadcast_to(w * loss, (NI, 128))[None]


def kernel(y_pred, y_true):
    ypt = jnp.transpose(y_pred, (1, 0, 2))
    ytt = jnp.transpose(y_true.reshape(B_IMG, G_PER, 6), (1, 0, 2))
    db = jnp.asarray(_DB_ALL)
    out = pl.pallas_call(
        _ssd_body,
        grid=(N_GRP,),
        in_specs=[
            pl.BlockSpec((25, NI, N_ANCHORS), lambda i: (0, i, 0)),
            pl.BlockSpec((G_PER, NI, 6), lambda i: (0, i, 0)),
            pl.BlockSpec((G_PER, N_ANCHORS), lambda i: (0, 0)),
        ],
        out_specs=pl.BlockSpec((1, NI, 128), lambda i: (i, 0, 0)),
        out_shape=jax.ShapeDtypeStruct((N_GRP, NI, 128), jnp.float32),
        compiler_params=pltpu.CompilerParams(
            dimension_semantics=("parallel",)),
    )(ypt, ytt, db)
    return jnp.sum(out[:, :, 0])


# speedup vs baseline: 1.8074x; 1.0633x over previous
import jax
import jax.numpy as jnp
import numpy as np
from math import sqrt
from jax.experimental import pallas as pl
from jax.experimental.pallas import tpu as pltpu

_MAPS = (38, 19, 10, 5, 3, 1)
_NA = (4, 6, 6, 6, 4, 4)
_RATIOS = (1, 2, 3)
_SCALES = (21, 45, 99, 153, 207, 261, 315)

N_ANCHORS = 8732
N_CLASSES = 21
B_IMG = 32
G_PER = 8
NI = 16
N_GRP = B_IMG // NI


def _build_default_boxes():
    m = 6
    fig = 300.0
    sb = np.array(_SCALES, dtype=np.float64) / fig
    sbh = [sqrt(sb[k] * sb[k + 1]) for k in range(m)]
    sb6 = sb[:m]
    whs = []
    for r in _RATIOS:
        rs = sqrt(r)
        if rs == 1.0:
            whs += [np.array([sb6, sb6]), np.array([sbh, sbh])]
        else:
            w = sb6 * rs
            h = sb6 / rs
            whs += [np.array([w, h]), np.array([h, w])]
    whs = np.array(whs).transpose([2, 1, 0]).clip(0, 1)
    mm = max(_MAPS)
    ax = np.arange(mm, dtype=np.float64) + 0.5
    cx = np.broadcast_to(ax.reshape(1, mm), (mm, mm))
    cy = np.broadcast_to(ax.reshape(mm, 1), (mm, mm))
    org = np.stack([cx, cy, np.ones((mm, mm)), np.ones((mm, mm))], 0)
    parts = []
    for na, ms, bw in zip(_NA, _MAPS, whs):
        fm = org[:, :ms, :ms].reshape(4, -1).copy()
        fm[:2] /= ms
        for idx in range(na):
            f = fm.copy()
            f[-2:] = bw[:, idx:idx + 1]
            parts.append(f)
    xywh = np.clip(np.concatenate(parts, axis=-1), 0, 1)
    x, y, w, h = xywh
    ltrb = np.stack([x - w / 2.0, y - h / 2.0, x + w / 2.0, y + h / 2.0])
    return np.concatenate([xywh, ltrb], axis=0).astype(np.float32)

_DB_ALL = _build_default_boxes()


def _ssd_body(pred_ref, yt_ref, db_ref, out_ref):
    i = pl.program_id(0)
    pred = pred_ref[...]
    yt = yt_ref[...]
    db = db_ref[...]

    dbx, dby, dbw, dbh = db[0:1], db[1:2], db[2:3], db[3:4]
    dl, dt, dr, dbt = db[4:5], db[5:6], db[6:7], db[7:8]
    a1 = (dr - dl) * (dbt - dt)
    lane = jax.lax.broadcasted_iota(jnp.int32, (NI, N_ANCHORS), 1)

    best = None
    gcls = gcx = gcy = gw = gh = None
    for g in range(G_PER):
        ytg = yt[g]
        cx, cy = ytg[:, 2:3], ytg[:, 3:4]
        w, h = ytg[:, 4:5], ytg[:, 5:6]
        gl, gr = cx - w * 0.5, cx + w * 0.5
        gt_, gb = cy - h * 0.5, cy + h * 0.5
        il = jnp.maximum(dl, gl)
        it = jnp.minimum(dr, gr)
        inter = (jnp.maximum(it - il, 0.0) *
                 jnp.maximum(jnp.minimum(dbt, gb) - jnp.maximum(dt, gt_), 0.0))
        a2 = (gr - gl) * (gb - gt_)
        iou = inter / (a1 + a2 - inter)
        rowmax = jnp.max(iou, axis=1, keepdims=True)
        dbox = jnp.min(jnp.where(iou == rowmax, lane, jnp.int32(2**30)),
                       axis=1, keepdims=True)
        iou = jnp.where(lane == dbox, 2.0, iou)
        if g == 0:
            best = iou
            full = (NI, N_ANCHORS)
            gcls = jnp.broadcast_to(ytg[:, 1:2], full)
            gcx = jnp.broadcast_to(cx, full)
            gcy = jnp.broadcast_to(cy, full)
            gw = jnp.broadcast_to(w, full)
            gh = jnp.broadcast_to(h, full)
        else:
            take = iou > best
            best = jnp.where(take, iou, best)
            gcls = jnp.where(take, ytg[:, 1:2], gcls)
            gcx = jnp.where(take, cx, gcx)
            gcy = jnp.where(take, cy, gcy)
            gw = jnp.where(take, w, gw)
            gh = jnp.where(take, h, gh)
    pos_mask = best > 0.5
    posf = pos_mask.astype(jnp.float32)
    n_pos = jnp.sum(posf, axis=1, keepdims=True)

    hats = ((gcx - dbx) / dbw, (gcy - dby) / dbh,
            jnp.log(gw / dbw), jnp.log(gh / dbh))
    loc_sum = jnp.zeros((NI, 1), jnp.float32)
    for r in range(4):
        d = jnp.abs(pred[r] - hats[r])
        sl = jnp.where(d < 1.0, 0.5 * d * d, d - 0.5)
        loc_sum = loc_sum + jnp.sum(sl * posf, axis=1, keepdims=True)
    loc_loss = loc_sum / (4.0 * n_pos)

    x = pred[4:25]
    xmax = jnp.max(x, axis=0)
    sumexp = jnp.sum(jnp.exp(x - xmax[None]), axis=0)
    lse = xmax + jnp.log(sumexp)

    gci = gcls.astype(jnp.int32)
    m0 = (gci & 1) > 0
    m1 = (gci & 2) > 0
    m2 = (gci & 4) > 0
    m3 = (gci & 8) > 0
    m4 = (gci & 16) > 0
    t = [jnp.where(m0, pred[4 + 2 * j + 1], pred[4 + 2 * j])
         for j in range(10)] + [pred[24]]
    u = [jnp.where(m1, t[2 * j + 1], t[2 * j]) for j in range(5)] + [t[10]]
    v = [jnp.where(m2, u[2 * j + 1], u[2 * j]) for j in range(3)]
    w0 = jnp.where(m3, v[1], v[0])
    xg = jnp.where(m4, v[2], w0)
    conf_pos_mean = (jnp.sum(jnp.where(pos_mask, lse - xg, 0.0),
                             axis=1, keepdims=True) / n_pos)

    neg = lse - pred[4]
    cand = jnp.where(pos_mask, -1.0, neg)
    vb = jax.lax.bitcast_convert_type(cand, jnp.int32)
    m_i = 3 * jnp.sum(pos_mask.astype(jnp.int32), axis=1, keepdims=True)
    hi0 = jnp.max(vb, axis=1, keepdims=True) + 1

    def search_body(_, carry):
        lo, hi = carry
        mid = lo + (hi - lo) // 2
        cnt = jnp.sum((vb >= mid).astype(jnp.int32), axis=1, keepdims=True)
        ge = cnt >= m_i
        return jnp.where(ge, mid, lo), jnp.where(ge, hi, mid)

    lo_f, _ = jax.lax.fori_loop(
        0, 32, search_body, (jnp.zeros((NI, 1), jnp.int32), hi0))
    vm = jax.lax.bitcast_convert_type(lo_f, jnp.float32)
    gt_mask = vb > lo_f
    c_gt = jnp.sum(gt_mask.astype(jnp.float32), axis=1, keepdims=True)
    s_gt = jnp.sum(jnp.where(gt_mask, cand, 0.0), axis=1, keepdims=True)
    mf = m_i.astype(jnp.float32)
    top_mean = (s_gt + (mf - c_gt) * vm) / mf

    loss = loc_loss + conf_pos_mean + top_mean
    img = (i * NI + jax.lax.broadcasted_iota(jnp.int32, (NI, 1), 0))
    w = img.astype(jnp.float32) * (1.0 / B_IMG)
    out_ref[...] = jnp.broadcast_to(w * loss, (NI, 128))[None]


def kernel(y_pred, y_true):
    ypt = jnp.transpose(y_pred, (1, 0, 2))
    ytt = jnp.transpose(y_true.reshape(B_IMG, G_PER, 6), (1, 0, 2))
    db = jnp.asarray(_DB_ALL)
    out = pl.pallas_call(
        _ssd_body,
        grid=(N_GRP,),
        in_specs=[
            pl.BlockSpec((25, NI, N_ANCHORS), lambda i: (0, i, 0)),
            pl.BlockSpec((G_PER, NI, 6), lambda i: (0, i, 0)),
            pl.BlockSpec((G_PER, N_ANCHORS), lambda i: (0, 0)),
        ],
        out_specs=pl.BlockSpec((1, NI, 128), lambda i: (i, 0, 0)),
        out_shape=jax.ShapeDtypeStruct((N_GRP, NI, 128), jnp.float32),
        compiler_params=pltpu.CompilerParams(
            dimension_semantics=("parallel",)),
    )(ypt, ytt, db)
    return jnp.sum(out[:, :, 0])
